# baseline (speedup 1.0000x reference)
import functools

import jax
import jax.numpy as jnp
from jax import lax
from jax.experimental import pallas as pl
from jax.experimental.pallas import tpu as pltpu

_THRESH = 0.9
_MIN_KEPT = 131072
_IGNORE = -1

_BH = 64
_RS = 8
_BR = 512


def _pixel_stats(pred_ref, tgt_ref, w_ref, r0, rs, *, nc):
    t = tgt_ref[0, r0:r0 + rs]
    valid = t != _IGNORE
    tt = jnp.where(valid, t, 0)

    m = pred_ref[0, 0, r0:r0 + rs]
    for ci in range(1, nc):
        m = jnp.maximum(m, pred_ref[0, ci, r0:r0 + rs])

    s = jnp.zeros_like(m)
    tl = jnp.zeros_like(m)
    wt = jnp.zeros_like(m)
    for ci in range(nc):
        xc = pred_ref[0, ci, r0:r0 + rs]
        s = s + jnp.exp(xc - m)
        hit = tt == ci
        tl = jnp.where(hit, xc, tl)
        wt = jnp.where(hit, w_ref[ci], wt)

    lse = jnp.log(s) + m
    nll = lse - tl
    p = jnp.exp(tl - lse)
    loss = jnp.where(valid, nll * wt, 0.0)
    return valid, p, loss


def _stats_kernel(pred_ref, tgt_ref, w_ref, acc_ref, *, nc):
    i = pl.program_id(0)
    j = pl.program_id(1)

    del i
    @pl.when(j == 0)
    def _():
        acc_ref[...] = jnp.zeros_like(acc_ref)

    c09 = jnp.float32(0.0)
    s09 = jnp.float32(0.0)
    nv = jnp.float32(0.0)
    for r0 in range(0, _BH, _RS):
        valid, p, loss = _pixel_stats(pred_ref, tgt_ref, w_ref, r0, _RS,
                                      nc=nc)
        sel = jnp.logical_and(valid, p < _THRESH)
        c09 = c09 + jnp.sum(jnp.where(sel, 1.0, 0.0))
        s09 = s09 + jnp.sum(jnp.where(sel, loss, 0.0))
        nv = nv + jnp.sum(jnp.where(valid, 1.0, 0.0))

    row = lax.broadcasted_iota(jnp.int32, acc_ref.shape, 1)
    lane = lax.broadcasted_iota(jnp.int32, acc_ref.shape, 2)
    vec = jnp.where(lane == 0, c09, jnp.where(lane == 1, s09,
                    jnp.where(lane == 2, nv, 0.0)))
    vec = jnp.where(row == 0, vec, 0.0)
    acc_ref[...] = acc_ref[...] + vec


def _ploss_kernel(pred_ref, tgt_ref, w_ref, p_ref, loss_ref, *, nc):
    for r0 in range(0, _BH, _RS):
        valid, p, loss = _pixel_stats(pred_ref, tgt_ref, w_ref, r0, _RS,
                                      nc=nc)
        p_ref[0, r0:r0 + _RS] = jnp.where(valid, p, jnp.inf)
        loss_ref[0, r0:r0 + _RS] = loss


def _hist_kernel(pref_ref, p_ref, hist_ref, *, shift):
    @pl.when(pl.program_id(0) == 0)
    def _():
        hist_ref[...] = jnp.zeros_like(hist_ref)

    bits = lax.bitcast_convert_type(p_ref[...], jnp.int32)
    dig = (bits >> shift) & 0xF
    if shift == 28:
        ok = jnp.full(dig.shape, True)
    else:
        ok = (bits >> (shift + 4)) == pref_ref[0]

    lane = lax.broadcasted_iota(jnp.int32, hist_ref.shape, 1)
    vec = jnp.zeros(hist_ref.shape, jnp.float32)
    for b in range(16):
        cnt = jnp.sum(jnp.where(jnp.logical_and(ok, dig == b), 1.0, 0.0))
        vec = vec + jnp.where(lane == b, cnt, 0.0)
    hist_ref[...] = hist_ref[...] + vec


def _sel_kernel(thr_ref, p_ref, loss_ref, out_ref):
    @pl.when(pl.program_id(0) == 0)
    def _():
        out_ref[...] = jnp.zeros_like(out_ref)

    sel = p_ref[...] < thr_ref[0]
    num = jnp.sum(jnp.where(sel, loss_ref[...], 0.0))
    den = jnp.sum(jnp.where(sel, 1.0, 0.0))
    lane = lax.broadcasted_iota(jnp.int32, out_ref.shape, 1)
    vec = jnp.where(lane == 0, num, jnp.where(lane == 1, den, 0.0))
    out_ref[...] = out_ref[...] + vec


@jax.jit
def kernel(predict, target, weight):
    n, nc, h, w = predict.shape

    acc = pl.pallas_call(
        functools.partial(_stats_kernel, nc=nc),
        grid=(n, h // _BH),
        in_specs=[
            pl.BlockSpec((1, nc, _BH, w), lambda i, j: (i, 0, j, 0)),
            pl.BlockSpec((1, _BH, w), lambda i, j: (i, j, 0)),
            pl.BlockSpec(memory_space=pltpu.SMEM),
        ],
        out_specs=pl.BlockSpec((1, 8, 128), lambda i, j: (i, 0, 0)),
        out_shape=jax.ShapeDtypeStruct((n, 8, 128), jnp.float32),
        compiler_params=pltpu.CompilerParams(
            dimension_semantics=("parallel", "arbitrary")),
    )(predict, target, weight)

    accs = jnp.sum(acc[:, 0, :], axis=0)
    n_valid = accs[2].astype(jnp.int32)
    k = jnp.minimum(_MIN_KEPT, n_valid - 1)
    c09 = accs[0]
    s09 = accs[1]
    nrows = n * h

    def _easy(_):
        return s09 / c09

    def _hard(_):
        p_arr, loss_arr = pl.pallas_call(
            functools.partial(_ploss_kernel, nc=nc),
            grid=(n, h // _BH),
            in_specs=[
                pl.BlockSpec((1, nc, _BH, w), lambda i, j: (i, 0, j, 0)),
                pl.BlockSpec((1, _BH, w), lambda i, j: (i, j, 0)),
                pl.BlockSpec(memory_space=pltpu.SMEM),
            ],
            out_specs=[
                pl.BlockSpec((1, _BH, w), lambda i, j: (i, j, 0)),
                pl.BlockSpec((1, _BH, w), lambda i, j: (i, j, 0)),
            ],
            out_shape=[
                jax.ShapeDtypeStruct((n, h, w), jnp.float32),
                jax.ShapeDtypeStruct((n, h, w), jnp.float32),
            ],
        )(predict, target, weight)
        p2d = p_arr.reshape(nrows, w)
        loss2d = loss_arr.reshape(nrows, w)
        prefix = jnp.int32(0)
        krem = k
        for l in range(8):
            shift = 28 - 4 * l
            hist = pl.pallas_call(
                functools.partial(_hist_kernel, shift=shift),
                grid=(nrows // _BR,),
                in_specs=[
                    pl.BlockSpec(memory_space=pltpu.SMEM),
                    pl.BlockSpec((_BR, w), lambda i: (i, 0)),
                ],
                out_specs=pl.BlockSpec((1, 128), lambda i: (0, 0)),
                out_shape=jax.ShapeDtypeStruct((1, 128), jnp.float32),
            )(prefix.reshape(1), p2d)
            h16 = hist[0, :16].astype(jnp.int32)
            cum = jnp.cumsum(h16)
            b = jnp.argmax(cum > krem).astype(jnp.int32)
            krem = krem - (cum[b] - h16[b])
            prefix = (prefix << 4) | b

        min_thr = lax.bitcast_convert_type(prefix, jnp.float32)
        threshold = jnp.maximum(min_thr, jnp.float32(_THRESH))

        sums = pl.pallas_call(
            _sel_kernel,
            grid=(nrows // _BR,),
            in_specs=[
                pl.BlockSpec(memory_space=pltpu.SMEM),
                pl.BlockSpec((_BR, w), lambda i: (i, 0)),
                pl.BlockSpec((_BR, w), lambda i: (i, 0)),
            ],
            out_specs=pl.BlockSpec((1, 128), lambda i: (0, 0)),
            out_shape=jax.ShapeDtypeStruct((1, 128), jnp.float32),
        )(threshold.reshape(1), p2d, loss2d)
        return sums[0, 0] / sums[0, 1]

    return lax.cond(c09 > k.astype(jnp.float32), _easy, _hard, 0)


# speedup vs baseline: 1.2699x; 1.2699x over previous
import functools

import jax
import jax.numpy as jnp
from jax import lax
from jax.experimental import pallas as pl
from jax.experimental.pallas import tpu as pltpu

_THRESH = 0.9
_MIN_KEPT = 131072
_IGNORE = -1

_BH = 128
_RS = 8
_BR = 512


def _pixel_stats(pred_ref, tgt_ref, w_ref, r0, rs, *, nc):
    t = tgt_ref[0, r0:r0 + rs]
    valid = t != _IGNORE
    tt = jnp.where(valid, t, 0)

    m = pred_ref[0, 0, r0:r0 + rs]
    for ci in range(1, nc):
        m = jnp.maximum(m, pred_ref[0, ci, r0:r0 + rs])

    s = jnp.zeros_like(m)
    tl = jnp.zeros_like(m)
    wt = jnp.zeros_like(m)
    for ci in range(nc):
        xc = pred_ref[0, ci, r0:r0 + rs]
        s = s + jnp.exp(xc - m)
        hit = tt == ci
        tl = jnp.where(hit, xc, tl)
        wt = jnp.where(hit, w_ref[ci], wt)

    lse = jnp.log(s) + m
    nll = lse - tl
    p = jnp.exp(tl - lse)
    loss = jnp.where(valid, nll * wt, 0.0)
    return valid, p, loss


def _stats_kernel(pred_ref, tgt_ref, w_ref, acc_ref, *, nc):
    i = pl.program_id(0)
    j = pl.program_id(1)

    @pl.when(jnp.logical_and(i == 0, j == 0))
    def _():
        acc_ref[...] = jnp.zeros_like(acc_ref)

    c09 = jnp.float32(0.0)
    s09 = jnp.float32(0.0)
    nv = jnp.float32(0.0)
    for r0 in range(0, _BH, _RS):
        valid, p, loss = _pixel_stats(pred_ref, tgt_ref, w_ref, r0, _RS,
                                      nc=nc)
        sel = jnp.logical_and(valid, p < _THRESH)
        c09 = c09 + jnp.sum(jnp.where(sel, 1.0, 0.0))
        s09 = s09 + jnp.sum(jnp.where(sel, loss, 0.0))
        nv = nv + jnp.sum(jnp.where(valid, 1.0, 0.0))

    lane = lax.broadcasted_iota(jnp.int32, acc_ref.shape, 1)
    vec = jnp.where(lane == 0, c09, jnp.where(lane == 1, s09,
                    jnp.where(lane == 2, nv, 0.0)))
    acc_ref[...] = acc_ref[...] + vec


def _ploss_kernel(pred_ref, tgt_ref, w_ref, p_ref, loss_ref, *, nc):
    for r0 in range(0, _BH, _RS):
        valid, p, loss = _pixel_stats(pred_ref, tgt_ref, w_ref, r0, _RS,
                                      nc=nc)
        p_ref[0, r0:r0 + _RS] = jnp.where(valid, p, jnp.inf)
        loss_ref[0, r0:r0 + _RS] = loss


def _hist_kernel(pref_ref, p_ref, hist_ref, *, shift):
    @pl.when(pl.program_id(0) == 0)
    def _():
        hist_ref[...] = jnp.zeros_like(hist_ref)

    bits = lax.bitcast_convert_type(p_ref[...], jnp.int32)
    dig = (bits >> shift) & 0xF
    if shift == 28:
        ok = jnp.full(dig.shape, True)
    else:
        ok = (bits >> (shift + 4)) == pref_ref[0]

    lane = lax.broadcasted_iota(jnp.int32, hist_ref.shape, 1)
    vec = jnp.zeros(hist_ref.shape, jnp.float32)
    for b in range(16):
        cnt = jnp.sum(jnp.where(jnp.logical_and(ok, dig == b), 1.0, 0.0))
        vec = vec + jnp.where(lane == b, cnt, 0.0)
    hist_ref[...] = hist_ref[...] + vec


def _sel_kernel(thr_ref, p_ref, loss_ref, out_ref):
    @pl.when(pl.program_id(0) == 0)
    def _():
        out_ref[...] = jnp.zeros_like(out_ref)

    sel = p_ref[...] < thr_ref[0]
    num = jnp.sum(jnp.where(sel, loss_ref[...], 0.0))
    den = jnp.sum(jnp.where(sel, 1.0, 0.0))
    lane = lax.broadcasted_iota(jnp.int32, out_ref.shape, 1)
    vec = jnp.where(lane == 0, num, jnp.where(lane == 1, den, 0.0))
    out_ref[...] = out_ref[...] + vec


@jax.jit
def kernel(predict, target, weight):
    n, nc, h, w = predict.shape

    acc = pl.pallas_call(
        functools.partial(_stats_kernel, nc=nc),
        grid=(n, h // _BH),
        in_specs=[
            pl.BlockSpec((1, nc, _BH, w), lambda i, j: (i, 0, j, 0)),
            pl.BlockSpec((1, _BH, w), lambda i, j: (i, j, 0)),
            pl.BlockSpec(memory_space=pltpu.SMEM),
        ],
        out_specs=pl.BlockSpec((1, 128), lambda i, j: (0, 0)),
        out_shape=jax.ShapeDtypeStruct((1, 128), jnp.float32),
    )(predict, target, weight)

    n_valid = acc[0, 2].astype(jnp.int32)
    k = jnp.minimum(_MIN_KEPT, n_valid - 1)
    c09 = acc[0, 0]
    s09 = acc[0, 1]
    nrows = n * h

    def _easy(_):
        return s09 / c09

    def _hard(_):
        p_arr, loss_arr = pl.pallas_call(
            functools.partial(_ploss_kernel, nc=nc),
            grid=(n, h // _BH),
            in_specs=[
                pl.BlockSpec((1, nc, _BH, w), lambda i, j: (i, 0, j, 0)),
                pl.BlockSpec((1, _BH, w), lambda i, j: (i, j, 0)),
                pl.BlockSpec(memory_space=pltpu.SMEM),
            ],
            out_specs=[
                pl.BlockSpec((1, _BH, w), lambda i, j: (i, j, 0)),
                pl.BlockSpec((1, _BH, w), lambda i, j: (i, j, 0)),
            ],
            out_shape=[
                jax.ShapeDtypeStruct((n, h, w), jnp.float32),
                jax.ShapeDtypeStruct((n, h, w), jnp.float32),
            ],
        )(predict, target, weight)
        p2d = p_arr.reshape(nrows, w)
        loss2d = loss_arr.reshape(nrows, w)
        prefix = jnp.int32(0)
        krem = k
        for l in range(8):
            shift = 28 - 4 * l
            hist = pl.pallas_call(
                functools.partial(_hist_kernel, shift=shift),
                grid=(nrows // _BR,),
                in_specs=[
                    pl.BlockSpec(memory_space=pltpu.SMEM),
                    pl.BlockSpec((_BR, w), lambda i: (i, 0)),
                ],
                out_specs=pl.BlockSpec((1, 128), lambda i: (0, 0)),
                out_shape=jax.ShapeDtypeStruct((1, 128), jnp.float32),
            )(prefix.reshape(1), p2d)
            h16 = hist[0, :16].astype(jnp.int32)
            cum = jnp.cumsum(h16)
            b = jnp.argmax(cum > krem).astype(jnp.int32)
            krem = krem - (cum[b] - h16[b])
            prefix = (prefix << 4) | b

        min_thr = lax.bitcast_convert_type(prefix, jnp.float32)
        threshold = jnp.maximum(min_thr, jnp.float32(_THRESH))

        sums = pl.pallas_call(
            _sel_kernel,
            grid=(nrows // _BR,),
            in_specs=[
                pl.BlockSpec(memory_space=pltpu.SMEM),
                pl.BlockSpec((_BR, w), lambda i: (i, 0)),
                pl.BlockSpec((_BR, w), lambda i: (i, 0)),
            ],
            out_specs=pl.BlockSpec((1, 128), lambda i: (0, 0)),
            out_shape=jax.ShapeDtypeStruct((1, 128), jnp.float32),
        )(threshold.reshape(1), p2d, loss2d)
        return sums[0, 0] / sums[0, 1]

    return lax.cond(c09 > k.astype(jnp.float32), _easy, _hard, 0)


# speedup vs baseline: 1.4305x; 1.1265x over previous
import functools

import jax
import jax.numpy as jnp
from jax import lax
from jax.experimental import pallas as pl
from jax.experimental.pallas import tpu as pltpu

_THRESH = 0.9
_MIN_KEPT = 131072
_IGNORE = -1

_BH = 256
_RS = 8
_BR = 512


def _pixel_stats(pred_ref, tgt_ref, w_ref, r0, rs, *, nc):
    t = tgt_ref[0, r0:r0 + rs]
    valid = t != _IGNORE
    tt = jnp.where(valid, t, 0)

    m = pred_ref[0, 0, r0:r0 + rs]
    for ci in range(1, nc):
        m = jnp.maximum(m, pred_ref[0, ci, r0:r0 + rs])

    s = jnp.zeros_like(m)
    tl = jnp.zeros_like(m)
    wt = jnp.zeros_like(m)
    for ci in range(nc):
        xc = pred_ref[0, ci, r0:r0 + rs]
        s = s + jnp.exp(xc - m)
        hit = tt == ci
        tl = jnp.where(hit, xc, tl)
        wt = jnp.where(hit, w_ref[ci], wt)

    lse = jnp.log(s) + m
    nll = lse - tl
    p = jnp.exp(tl - lse)
    loss = jnp.where(valid, nll * wt, 0.0)
    return valid, p, loss


def _stats_kernel(pred_ref, tgt_ref, w_ref, acc_ref, *, nc):
    i = pl.program_id(0)
    j = pl.program_id(1)

    @pl.when(jnp.logical_and(i == 0, j == 0))
    def _():
        acc_ref[...] = jnp.zeros_like(acc_ref)

    c09 = jnp.float32(0.0)
    s09 = jnp.float32(0.0)
    nv = jnp.float32(0.0)
    for r0 in range(0, _BH, _RS):
        valid, p, loss = _pixel_stats(pred_ref, tgt_ref, w_ref, r0, _RS,
                                      nc=nc)
        sel = jnp.logical_and(valid, p < _THRESH)
        c09 = c09 + jnp.sum(jnp.where(sel, 1.0, 0.0))
        s09 = s09 + jnp.sum(jnp.where(sel, loss, 0.0))
        nv = nv + jnp.sum(jnp.where(valid, 1.0, 0.0))

    lane = lax.broadcasted_iota(jnp.int32, acc_ref.shape, 1)
    vec = jnp.where(lane == 0, c09, jnp.where(lane == 1, s09,
                    jnp.where(lane == 2, nv, 0.0)))
    acc_ref[...] = acc_ref[...] + vec


def _ploss_kernel(pred_ref, tgt_ref, w_ref, p_ref, loss_ref, *, nc):
    for r0 in range(0, _BH, _RS):
        valid, p, loss = _pixel_stats(pred_ref, tgt_ref, w_ref, r0, _RS,
                                      nc=nc)
        p_ref[0, r0:r0 + _RS] = jnp.where(valid, p, jnp.inf)
        loss_ref[0, r0:r0 + _RS] = loss


def _hist_kernel(pref_ref, p_ref, hist_ref, *, shift):
    @pl.when(pl.program_id(0) == 0)
    def _():
        hist_ref[...] = jnp.zeros_like(hist_ref)

    bits = lax.bitcast_convert_type(p_ref[...], jnp.int32)
    dig = (bits >> shift) & 0xF
    if shift == 28:
        ok = jnp.full(dig.shape, True)
    else:
        ok = (bits >> (shift + 4)) == pref_ref[0]

    lane = lax.broadcasted_iota(jnp.int32, hist_ref.shape, 1)
    vec = jnp.zeros(hist_ref.shape, jnp.float32)
    for b in range(16):
        cnt = jnp.sum(jnp.where(jnp.logical_and(ok, dig == b), 1.0, 0.0))
        vec = vec + jnp.where(lane == b, cnt, 0.0)
    hist_ref[...] = hist_ref[...] + vec


def _sel_kernel(thr_ref, p_ref, loss_ref, out_ref):
    @pl.when(pl.program_id(0) == 0)
    def _():
        out_ref[...] = jnp.zeros_like(out_ref)

    sel = p_ref[...] < thr_ref[0]
    num = jnp.sum(jnp.where(sel, loss_ref[...], 0.0))
    den = jnp.sum(jnp.where(sel, 1.0, 0.0))
    lane = lax.broadcasted_iota(jnp.int32, out_ref.shape, 1)
    vec = jnp.where(lane == 0, num, jnp.where(lane == 1, den, 0.0))
    out_ref[...] = out_ref[...] + vec


@jax.jit
def kernel(predict, target, weight):
    n, nc, h, w = predict.shape

    acc = pl.pallas_call(
        functools.partial(_stats_kernel, nc=nc),
        grid=(n, h // _BH),
        in_specs=[
            pl.BlockSpec((1, nc, _BH, w), lambda i, j: (i, 0, j, 0)),
            pl.BlockSpec((1, _BH, w), lambda i, j: (i, j, 0)),
            pl.BlockSpec(memory_space=pltpu.SMEM),
        ],
        out_specs=pl.BlockSpec((1, 128), lambda i, j: (0, 0)),
        out_shape=jax.ShapeDtypeStruct((1, 128), jnp.float32),
    )(predict, target, weight)

    n_valid = acc[0, 2].astype(jnp.int32)
    k = jnp.minimum(_MIN_KEPT, n_valid - 1)
    c09 = acc[0, 0]
    s09 = acc[0, 1]
    nrows = n * h

    def _easy(_):
        return s09 / c09

    def _hard(_):
        p_arr, loss_arr = pl.pallas_call(
            functools.partial(_ploss_kernel, nc=nc),
            grid=(n, h // _BH),
            in_specs=[
                pl.BlockSpec((1, nc, _BH, w), lambda i, j: (i, 0, j, 0)),
                pl.BlockSpec((1, _BH, w), lambda i, j: (i, j, 0)),
                pl.BlockSpec(memory_space=pltpu.SMEM),
            ],
            out_specs=[
                pl.BlockSpec((1, _BH, w), lambda i, j: (i, j, 0)),
                pl.BlockSpec((1, _BH, w), lambda i, j: (i, j, 0)),
            ],
            out_shape=[
                jax.ShapeDtypeStruct((n, h, w), jnp.float32),
                jax.ShapeDtypeStruct((n, h, w), jnp.float32),
            ],
        )(predict, target, weight)
        p2d = p_arr.reshape(nrows, w)
        loss2d = loss_arr.reshape(nrows, w)
        prefix = jnp.int32(0)
        krem = k
        for l in range(8):
            shift = 28 - 4 * l
            hist = pl.pallas_call(
                functools.partial(_hist_kernel, shift=shift),
                grid=(nrows // _BR,),
                in_specs=[
                    pl.BlockSpec(memory_space=pltpu.SMEM),
                    pl.BlockSpec((_BR, w), lambda i: (i, 0)),
                ],
                out_specs=pl.BlockSpec((1, 128), lambda i: (0, 0)),
                out_shape=jax.ShapeDtypeStruct((1, 128), jnp.float32),
            )(prefix.reshape(1), p2d)
            h16 = hist[0, :16].astype(jnp.int32)
            cum = jnp.cumsum(h16)
            b = jnp.argmax(cum > krem).astype(jnp.int32)
            krem = krem - (cum[b] - h16[b])
            prefix = (prefix << 4) | b

        min_thr = lax.bitcast_convert_type(prefix, jnp.float32)
        threshold = jnp.maximum(min_thr, jnp.float32(_THRESH))

        sums = pl.pallas_call(
            _sel_kernel,
            grid=(nrows // _BR,),
            in_specs=[
                pl.BlockSpec(memory_space=pltpu.SMEM),
                pl.BlockSpec((_BR, w), lambda i: (i, 0)),
                pl.BlockSpec((_BR, w), lambda i: (i, 0)),
            ],
            out_specs=pl.BlockSpec((1, 128), lambda i: (0, 0)),
            out_shape=jax.ShapeDtypeStruct((1, 128), jnp.float32),
        )(threshold.reshape(1), p2d, loss2d)
        return sums[0, 0] / sums[0, 1]

    return lax.cond(c09 > k.astype(jnp.float32), _easy, _hard, 0)


# speedup vs baseline: 1.4710x; 1.0283x over previous
import functools

import jax
import jax.numpy as jnp
from jax import lax
from jax.experimental import pallas as pl
from jax.experimental.pallas import tpu as pltpu

_THRESH = 0.9
_MIN_KEPT = 131072
_IGNORE = -1

_BH = 512
_RS = 8
_BR = 512


def _pixel_stats(pred_ref, tgt_ref, w_ref, r0, rs, *, nc):
    t = tgt_ref[0, r0:r0 + rs]
    valid = t != _IGNORE
    tt = jnp.where(valid, t, 0)

    m = pred_ref[0, 0, r0:r0 + rs]
    for ci in range(1, nc):
        m = jnp.maximum(m, pred_ref[0, ci, r0:r0 + rs])

    s = jnp.zeros_like(m)
    tl = jnp.zeros_like(m)
    wt = jnp.zeros_like(m)
    for ci in range(nc):
        xc = pred_ref[0, ci, r0:r0 + rs]
        s = s + jnp.exp(xc - m)
        hit = tt == ci
        tl = jnp.where(hit, xc, tl)
        wt = jnp.where(hit, w_ref[ci], wt)

    lse = jnp.log(s) + m
    nll = lse - tl
    p = jnp.exp(tl - lse)
    loss = jnp.where(valid, nll * wt, 0.0)
    return valid, p, loss


def _stats_kernel(pred_ref, tgt_ref, w_ref, acc_ref, *, nc):
    i = pl.program_id(0)
    j = pl.program_id(1)

    @pl.when(jnp.logical_and(i == 0, j == 0))
    def _():
        acc_ref[...] = jnp.zeros_like(acc_ref)

    c09 = jnp.float32(0.0)
    s09 = jnp.float32(0.0)
    nv = jnp.float32(0.0)
    for r0 in range(0, _BH, _RS):
        valid, p, loss = _pixel_stats(pred_ref, tgt_ref, w_ref, r0, _RS,
                                      nc=nc)
        sel = jnp.logical_and(valid, p < _THRESH)
        c09 = c09 + jnp.sum(jnp.where(sel, 1.0, 0.0))
        s09 = s09 + jnp.sum(jnp.where(sel, loss, 0.0))
        nv = nv + jnp.sum(jnp.where(valid, 1.0, 0.0))

    lane = lax.broadcasted_iota(jnp.int32, acc_ref.shape, 1)
    vec = jnp.where(lane == 0, c09, jnp.where(lane == 1, s09,
                    jnp.where(lane == 2, nv, 0.0)))
    acc_ref[...] = acc_ref[...] + vec


def _ploss_kernel(pred_ref, tgt_ref, w_ref, p_ref, loss_ref, *, nc):
    for r0 in range(0, _BH, _RS):
        valid, p, loss = _pixel_stats(pred_ref, tgt_ref, w_ref, r0, _RS,
                                      nc=nc)
        p_ref[0, r0:r0 + _RS] = jnp.where(valid, p, jnp.inf)
        loss_ref[0, r0:r0 + _RS] = loss


def _hist_kernel(pref_ref, p_ref, hist_ref, *, shift):
    @pl.when(pl.program_id(0) == 0)
    def _():
        hist_ref[...] = jnp.zeros_like(hist_ref)

    bits = lax.bitcast_convert_type(p_ref[...], jnp.int32)
    dig = (bits >> shift) & 0xF
    if shift == 28:
        ok = jnp.full(dig.shape, True)
    else:
        ok = (bits >> (shift + 4)) == pref_ref[0]

    lane = lax.broadcasted_iota(jnp.int32, hist_ref.shape, 1)
    vec = jnp.zeros(hist_ref.shape, jnp.float32)
    for b in range(16):
        cnt = jnp.sum(jnp.where(jnp.logical_and(ok, dig == b), 1.0, 0.0))
        vec = vec + jnp.where(lane == b, cnt, 0.0)
    hist_ref[...] = hist_ref[...] + vec


def _sel_kernel(thr_ref, p_ref, loss_ref, out_ref):
    @pl.when(pl.program_id(0) == 0)
    def _():
        out_ref[...] = jnp.zeros_like(out_ref)

    sel = p_ref[...] < thr_ref[0]
    num = jnp.sum(jnp.where(sel, loss_ref[...], 0.0))
    den = jnp.sum(jnp.where(sel, 1.0, 0.0))
    lane = lax.broadcasted_iota(jnp.int32, out_ref.shape, 1)
    vec = jnp.where(lane == 0, num, jnp.where(lane == 1, den, 0.0))
    out_ref[...] = out_ref[...] + vec


@jax.jit
def kernel(predict, target, weight):
    n, nc, h, w = predict.shape

    acc = pl.pallas_call(
        functools.partial(_stats_kernel, nc=nc),
        grid=(n, h // _BH),
        in_specs=[
            pl.BlockSpec((1, nc, _BH, w), lambda i, j: (i, 0, j, 0)),
            pl.BlockSpec((1, _BH, w), lambda i, j: (i, j, 0)),
            pl.BlockSpec(memory_space=pltpu.SMEM),
        ],
        out_specs=pl.BlockSpec((1, 128), lambda i, j: (0, 0)),
        out_shape=jax.ShapeDtypeStruct((1, 128), jnp.float32),
    )(predict, target, weight)

    n_valid = acc[0, 2].astype(jnp.int32)
    k = jnp.minimum(_MIN_KEPT, n_valid - 1)
    c09 = acc[0, 0]
    s09 = acc[0, 1]
    nrows = n * h

    def _easy(_):
        return s09 / c09

    def _hard(_):
        p_arr, loss_arr = pl.pallas_call(
            functools.partial(_ploss_kernel, nc=nc),
            grid=(n, h // _BH),
            in_specs=[
                pl.BlockSpec((1, nc, _BH, w), lambda i, j: (i, 0, j, 0)),
                pl.BlockSpec((1, _BH, w), lambda i, j: (i, j, 0)),
                pl.BlockSpec(memory_space=pltpu.SMEM),
            ],
            out_specs=[
                pl.BlockSpec((1, _BH, w), lambda i, j: (i, j, 0)),
                pl.BlockSpec((1, _BH, w), lambda i, j: (i, j, 0)),
            ],
            out_shape=[
                jax.ShapeDtypeStruct((n, h, w), jnp.float32),
                jax.ShapeDtypeStruct((n, h, w), jnp.float32),
            ],
        )(predict, target, weight)
        p2d = p_arr.reshape(nrows, w)
        loss2d = loss_arr.reshape(nrows, w)
        prefix = jnp.int32(0)
        krem = k
        for l in range(8):
            shift = 28 - 4 * l
            hist = pl.pallas_call(
                functools.partial(_hist_kernel, shift=shift),
                grid=(nrows // _BR,),
                in_specs=[
                    pl.BlockSpec(memory_space=pltpu.SMEM),
                    pl.BlockSpec((_BR, w), lambda i: (i, 0)),
                ],
                out_specs=pl.BlockSpec((1, 128), lambda i: (0, 0)),
                out_shape=jax.ShapeDtypeStruct((1, 128), jnp.float32),
            )(prefix.reshape(1), p2d)
            h16 = hist[0, :16].astype(jnp.int32)
            cum = jnp.cumsum(h16)
            b = jnp.argmax(cum > krem).astype(jnp.int32)
            krem = krem - (cum[b] - h16[b])
            prefix = (prefix << 4) | b

        min_thr = lax.bitcast_convert_type(prefix, jnp.float32)
        threshold = jnp.maximum(min_thr, jnp.float32(_THRESH))

        sums = pl.pallas_call(
            _sel_kernel,
            grid=(nrows // _BR,),
            in_specs=[
                pl.BlockSpec(memory_space=pltpu.SMEM),
                pl.BlockSpec((_BR, w), lambda i: (i, 0)),
                pl.BlockSpec((_BR, w), lambda i: (i, 0)),
            ],
            out_specs=pl.BlockSpec((1, 128), lambda i: (0, 0)),
            out_shape=jax.ShapeDtypeStruct((1, 128), jnp.float32),
        )(threshold.reshape(1), p2d, loss2d)
        return sums[0, 0] / sums[0, 1]

    return lax.cond(c09 > k.astype(jnp.float32), _easy, _hard, 0)
